# Initial kernel scaffold; baseline (speedup 1.0000x reference)
import jax
import jax.numpy as jnp
from jax.experimental import pallas as pl


def kernel(features, xy_coords, extrinsics, W, b):
    raise NotImplementedError("write your pallas kernel here")


# speedup vs baseline: 3.4541x; 3.4541x over previous
import functools

import jax
import jax.numpy as jnp
from jax import lax
from jax.experimental import pallas as pl
from jax.experimental.pallas import tpu as pltpu
from jax.experimental.pallas import tpu_sc as plsc

V = 8
G = 16384
C = 128
HW = 16384
VG = V * G
NS = 16
GP = G // NS
NCH = GP // 128
HHW = HW // 2
NDUM = 16
ODUM = 128


def _adj_body(e_ref, a_ref, at_ref):
    E = e_ref[...]
    cam = []
    for j in range(3):
        cj = -(E[:, 0 * 4 + j] * E[:, 3]
               + E[:, 1 * 4 + j] * E[:, 7]
               + E[:, 2 * 4 + j] * E[:, 11])
        cam.append(cj)
    dist_sq = ((cam[0][:, None] - cam[0][None, :]) ** 2
               + (cam[1][:, None] - cam[1][None, :]) ** 2
               + (cam[2][:, None] - cam[2][None, :]) ** 2)
    r_i = lax.broadcasted_iota(jnp.int32, (V, V), 0)
    c_i = lax.broadcasted_iota(jnp.int32, (V, V), 1)
    diag = r_i == c_i
    neg_inf = jnp.float32(-jnp.inf)
    work = jnp.where(diag, neg_inf, -dist_sq)
    mask = jnp.zeros((V, V), jnp.float32)
    for _ in range(3):
        m = jnp.max(work, axis=1, keepdims=True)
        eq = work == m
        cand = jnp.where(eq, c_i, V)
        jstar = jnp.min(cand, axis=1, keepdims=True)
        sel = c_i == jstar
        mask = jnp.where(sel, 1.0, mask)
        work = jnp.where(sel, neg_inf, work)
    a_overlap = 1.0 / (1.0 + jnp.sqrt(dist_sq + 1e-6))
    A = a_overlap * mask
    A = jnp.where(diag, 1.0, A)
    deg = jnp.sum(A, axis=1, keepdims=True)
    deg_inv = 1.0 / (deg + jnp.where(deg == 0, 1.0, 0.0))
    a_ref[...] = A
    at_ref[...] = A * deg_inv


_adj_call = pl.pallas_call(
    _adj_body,
    out_shape=(jax.ShapeDtypeStruct((V, V), jnp.float32),
               jax.ShapeDtypeStruct((V, V), jnp.float32)),
)


_RB = 2048


def _mm_body(x_ref, w_ref, b_ref, y_ref):
    y_ref[...] = (jnp.dot(x_ref[...], w_ref[...],
                          preferred_element_type=jnp.float32) + b_ref[...])


_mm_call = pl.pallas_call(
    _mm_body,
    grid=(VG // _RB,),
    in_specs=[
        pl.BlockSpec((_RB, C), lambda i: (i, 0)),
        pl.BlockSpec((C, C), lambda i: (0, 0)),
        pl.BlockSpec((1, C), lambda i: (0, 0)),
    ],
    out_specs=pl.BlockSpec((_RB, C), lambda i: (i, 0)),
    out_shape=jax.ShapeDtypeStruct((VG, C), jnp.float32),
)


def _idx_body(x_ref, y_ref, idxr_ref, ow_ref):
    xi = jnp.clip(jnp.round(x_ref[...]), 0, 127).astype(jnp.int32)
    yi = jnp.clip(jnp.round(y_ref[...]), 0, 127).astype(jnp.int32)
    idx = yi * 128 + xi
    i_row = lax.broadcasted_iota(jnp.int32, (V, G), 0)
    g_col = lax.broadcasted_iota(jnp.int32, (V, G), 1)
    grid_dum = HHW + (g_col & (NDUM - 1))
    out_dum = VG + (g_col & (ODUM - 1))
    out_real = i_row * G + g_col
    for c in range(2):
        t = idx - c * HHW
        inr = jnp.logical_and(t >= 0, t < HHW)
        idxr_ref[c] = jnp.where(inr, t, grid_dum)
        ow_ref[c] = jnp.where(inr, out_real, out_dum)


_idx_call = pl.pallas_call(
    _idx_body,
    grid=(8,),
    in_specs=[pl.BlockSpec((8, G), lambda i: (i, 0)),
              pl.BlockSpec((8, G), lambda i: (i, 0))],
    out_specs=[pl.BlockSpec((2, 8, G), lambda i: (0, i, 0)),
               pl.BlockSpec((2, 8, G), lambda i: (0, i, 0))],
    out_shape=[jax.ShapeDtypeStruct((2, V * V, G), jnp.int32),
               jax.ShapeDtypeStruct((2, V * V, G), jnp.int32)],
)


_mesh = plsc.VectorSubcoreMesh(core_axis_name="c", subcore_axis_name="s")


@functools.partial(
    pl.kernel,
    mesh=_mesh,
    out_type=jax.ShapeDtypeStruct((VG + ODUM, C), jnp.float32),
    scratch_types=[
        pltpu.VMEM_SHARED((HHW + NDUM, C), jnp.float32),
        pltpu.VMEM((NCH, 128), jnp.int32),
        pltpu.VMEM((NCH, 128), jnp.int32),
        pltpu.VMEM((NCH, 128), jnp.int32),
        pltpu.VMEM((128, C), jnp.float32),
        pltpu.VMEM((128, C), jnp.float32),
        pltpu.VMEM((128, C), jnp.float32),
        pltpu.VMEM((128,), jnp.float32),
    ],
)
def _sc_aggregate(y_hbm, idxr_hbm, ow_hbm, at_hbm, out_hbm,
                  grid_sh, idxbuf, selfbuf, owbuf, ybuf, gbuf, zbuf, wbuf):
    c = lax.axis_index("c")
    s = lax.axis_index("s")
    pltpu.sync_copy(at_hbm, wbuf)

    zero16 = jnp.zeros((16,), jnp.float32)

    def _zrow(r, carry):
        for q in range(C // 16):
            zbuf[r, pl.ds(q * 16, 16)] = zero16
        return carry

    lax.fori_loop(0, 128, _zrow, 0)

    def _per_target(i, carry):
        def _zslice(j, cc):
            pltpu.sync_copy(zbuf, grid_sh.at[pl.ds((s * 4 + j) * 128, 128), :])
            return cc

        lax.fori_loop(0, HHW // (128 * NS), _zslice, 0)
        plsc.subcore_barrier()

        def _per_view(v, cc):
            w = wbuf[pl.ds(i * V + v, 16)][0]

            @pl.when(w != 0.0)
            def _():
                pltpu.sync_copy(
                    idxr_hbm.at[c, v * V + i, pl.ds(s * NCH, NCH), :], idxbuf)

                def _chunk(j, c2):
                    row0 = (v * 128 + s * NCH + j) * 128
                    pltpu.sync_copy(y_hbm.at[pl.ds(row0, 128), :], ybuf)

                    def _scale(r, c3):
                        for q in range(C // 16):
                            sl = pl.ds(q * 16, 16)
                            ybuf[r, sl] = ybuf[r, sl] * w
                        return c3

                    lax.fori_loop(0, 128, _scale, 0)
                    pltpu.sync_copy(ybuf, grid_sh.at[idxbuf.at[j]], add=True)
                    return c2

                lax.fori_loop(0, NCH, _chunk, 0)

            return cc

        lax.fori_loop(0, V, _per_view, 0)
        plsc.subcore_barrier()

        pltpu.sync_copy(idxr_hbm.at[c, i * V + i, pl.ds(s * NCH, NCH), :], selfbuf)
        pltpu.sync_copy(ow_hbm.at[c, i * V + i, pl.ds(s * NCH, NCH), :], owbuf)

        def _out(j, cc):
            pltpu.sync_copy(grid_sh.at[selfbuf.at[j]], gbuf)
            pltpu.sync_copy(gbuf, out_hbm.at[owbuf.at[j]])
            return cc

        lax.fori_loop(0, NCH, _out, 0)
        plsc.subcore_barrier()
        return carry

    lax.fori_loop(0, V, _per_target, 0)


def kernel(features, xy_coords, extrinsics, W, b):
    assert features.shape == (1, V, G, C)
    assert xy_coords.shape == (1, V, V, G, 2)

    A, At = _adj_call(extrinsics.reshape(V, 16))

    y = _mm_call(features.reshape(VG, C), W, b.reshape(1, C))

    xc = jnp.moveaxis(xy_coords.reshape(V * V, G, 2), -1, 0)
    idxr, ow = _idx_call(xc[0], xc[1])
    idxr4 = idxr.reshape(2, V * V, G // 128, 128)
    ow4 = ow.reshape(2, V * V, G // 128, 128)

    at128 = jnp.zeros((128,), jnp.float32).at[:V * V].set(At.reshape(V * V))
    out2 = _sc_aggregate(y, idxr4, ow4, at128)
    return (out2[:VG].reshape(1, V, G, C), A.reshape(1, V, V))


# speedup vs baseline: 4.7733x; 1.3819x over previous
import functools

import jax
import jax.numpy as jnp
from jax import lax
from jax.experimental import pallas as pl
from jax.experimental.pallas import tpu as pltpu
from jax.experimental.pallas import tpu_sc as plsc

V = 8
G = 16384
C = 128
HW = 16384
VG = V * G
NS = 16
GP = G // NS
NCH = GP // 128
HHW = HW // 2
NDUM = 16
ODUM = 128


def _adj_body(e_ref, a_ref, at_ref):
    E = e_ref[...]
    cam = []
    for j in range(3):
        cj = -(E[:, 0 * 4 + j] * E[:, 3]
               + E[:, 1 * 4 + j] * E[:, 7]
               + E[:, 2 * 4 + j] * E[:, 11])
        cam.append(cj)
    dist_sq = ((cam[0][:, None] - cam[0][None, :]) ** 2
               + (cam[1][:, None] - cam[1][None, :]) ** 2
               + (cam[2][:, None] - cam[2][None, :]) ** 2)
    r_i = lax.broadcasted_iota(jnp.int32, (V, V), 0)
    c_i = lax.broadcasted_iota(jnp.int32, (V, V), 1)
    diag = r_i == c_i
    neg_inf = jnp.float32(-jnp.inf)
    work = jnp.where(diag, neg_inf, -dist_sq)
    mask = jnp.zeros((V, V), jnp.float32)
    for _ in range(3):
        m = jnp.max(work, axis=1, keepdims=True)
        eq = work == m
        cand = jnp.where(eq, c_i, V)
        jstar = jnp.min(cand, axis=1, keepdims=True)
        sel = c_i == jstar
        mask = jnp.where(sel, 1.0, mask)
        work = jnp.where(sel, neg_inf, work)
    a_overlap = 1.0 / (1.0 + jnp.sqrt(dist_sq + 1e-6))
    A = a_overlap * mask
    A = jnp.where(diag, 1.0, A)
    deg = jnp.sum(A, axis=1, keepdims=True)
    deg_inv = 1.0 / (deg + jnp.where(deg == 0, 1.0, 0.0))
    a_ref[...] = A
    at_ref[...] = A * deg_inv


_adj_call = pl.pallas_call(
    _adj_body,
    out_shape=(jax.ShapeDtypeStruct((V, V), jnp.float32),
               jax.ShapeDtypeStruct((V, V), jnp.float32)),
)


_RB = 2048


def _mm_body(x_ref, w_ref, b_ref, y_ref):
    y_ref[...] = (jnp.dot(x_ref[...], w_ref[...],
                          preferred_element_type=jnp.float32) + b_ref[...])


_mm_call = pl.pallas_call(
    _mm_body,
    grid=(VG // _RB,),
    in_specs=[
        pl.BlockSpec((_RB, C), lambda i: (i, 0)),
        pl.BlockSpec((C, C), lambda i: (0, 0)),
        pl.BlockSpec((1, C), lambda i: (0, 0)),
    ],
    out_specs=pl.BlockSpec((_RB, C), lambda i: (i, 0)),
    out_shape=jax.ShapeDtypeStruct((VG, C), jnp.float32),
)


def _idx_body(x_ref, y_ref, idxr_ref, ow_ref):
    xi = jnp.clip(jnp.round(x_ref[...]), 0, 127).astype(jnp.int32)
    yi = jnp.clip(jnp.round(y_ref[...]), 0, 127).astype(jnp.int32)
    idx = yi * 128 + xi
    i_row = lax.broadcasted_iota(jnp.int32, (V, G), 0)
    g_col = lax.broadcasted_iota(jnp.int32, (V, G), 1)
    grid_dum = HHW + (g_col & (NDUM - 1))
    out_dum = VG + (g_col & (ODUM - 1))
    out_real = i_row * G + g_col
    for c in range(2):
        t = idx - c * HHW
        inr = jnp.logical_and(t >= 0, t < HHW)
        idxr_ref[c] = jnp.where(inr, t, grid_dum)
        ow_ref[c] = jnp.where(inr, out_real, out_dum)


_idx_call = pl.pallas_call(
    _idx_body,
    grid=(8,),
    in_specs=[pl.BlockSpec((8, G), lambda i: (i, 0)),
              pl.BlockSpec((8, G), lambda i: (i, 0))],
    out_specs=[pl.BlockSpec((2, 8, G), lambda i: (0, i, 0)),
               pl.BlockSpec((2, 8, G), lambda i: (0, i, 0))],
    out_shape=[jax.ShapeDtypeStruct((2, V * V, G), jnp.int32),
               jax.ShapeDtypeStruct((2, V * V, G), jnp.int32)],
)


_mesh = plsc.VectorSubcoreMesh(core_axis_name="c", subcore_axis_name="s")


_NB = 3
_ZR = 32


@functools.partial(
    pl.kernel,
    mesh=_mesh,
    out_type=jax.ShapeDtypeStruct((VG + ODUM, C), jnp.float32),
    scratch_types=[
        pltpu.VMEM_SHARED((HHW + NDUM, C), jnp.float32),
        pltpu.VMEM((NCH, 128), jnp.int32),
        pltpu.VMEM((NCH, 128), jnp.int32),
        pltpu.VMEM((NCH, 128), jnp.int32),
        pltpu.VMEM((_NB, 128, C), jnp.float32),
        pltpu.VMEM((_ZR, C), jnp.float32),
        pltpu.VMEM((128,), jnp.float32),
    ] + [pltpu.SemaphoreType.DMA] * (2 * _NB),
)
def _sc_aggregate(y_hbm, idxr_hbm, ow_hbm, at_hbm, out_hbm,
                  grid_sh, idxbuf, selfbuf, owbuf, buf, zbuf, wbuf, *sems):
    c = lax.axis_index("c")
    s = lax.axis_index("s")
    sem_l = sems[:_NB]
    sem_s = sems[_NB:]
    pltpu.sync_copy(at_hbm, wbuf)

    zero16 = jnp.zeros((16,), jnp.float32)

    def _zrow(r, carry):
        for q in range(C // 16):
            zbuf[r, pl.ds(q * 16, 16)] = zero16
        return carry

    lax.fori_loop(0, _ZR, _zrow, 0)

    def _per_target(i, carry):
        nz = HHW // (_ZR * NS)

        def _zslice(j, cc):
            pltpu.sync_copy(zbuf, grid_sh.at[pl.ds(s * (HHW // NS) + j * _ZR, _ZR), :])
            return cc

        lax.fori_loop(0, nz, _zslice, 0)
        plsc.subcore_barrier()

        def _per_view(v, cc):
            w = wbuf[pl.ds(i * V + v, 16)][0]

            @pl.when(w != 0.0)
            def _():
                pltpu.sync_copy(
                    idxr_hbm.at[c, v * V + i, pl.ds(s * NCH, NCH), :], idxbuf)

                def _load(j):
                    row0 = (v * 128 + s * NCH + j) * 128
                    return pltpu.async_copy(
                        y_hbm.at[pl.ds(row0, 128), :], buf.at[j % _NB],
                        sem_l[j % _NB])

                hl = [None] * NCH
                hs = [None] * NCH
                for j in range(_NB - 1):
                    hl[j] = _load(j)
                for j in range(NCH):
                    b = j % _NB
                    hl[j].wait()

                    def _scale(r, c3):
                        for q in range(C // 16):
                            sl = pl.ds(q * 16, 16)
                            buf[b, r, sl] = buf[b, r, sl] * w
                        return c3

                    lax.fori_loop(0, 128, _scale, 0)
                    hs[j] = pltpu.async_copy(
                        buf.at[b], grid_sh.at[idxbuf.at[j]], sem_s[b], add=True)
                    if j + _NB - 1 < NCH:
                        if j >= 1:
                            hs[j - 1].wait()
                        hl[j + _NB - 1] = _load(j + _NB - 1)
                for j in range(max(0, NCH - _NB), NCH):
                    if hs[j] is not None:
                        hs[j].wait()

            return cc

        lax.fori_loop(0, V, _per_view, 0)
        plsc.subcore_barrier()

        pltpu.sync_copy(idxr_hbm.at[c, i * V + i, pl.ds(s * NCH, NCH), :], selfbuf)
        pltpu.sync_copy(ow_hbm.at[c, i * V + i, pl.ds(s * NCH, NCH), :], owbuf)

        def _gload(j):
            return pltpu.async_copy(
                grid_sh.at[selfbuf.at[j]], buf.at[j % _NB], sem_l[j % _NB])

        hg = [None] * NCH
        ho = [None] * NCH
        for j in range(_NB - 1):
            hg[j] = _gload(j)
        for j in range(NCH):
            b = j % _NB
            hg[j].wait()
            ho[j] = pltpu.async_copy(buf.at[b], out_hbm.at[owbuf.at[j]], sem_s[b])
            if j + _NB - 1 < NCH:
                if j >= 1:
                    ho[j - 1].wait()
                hg[j + _NB - 1] = _gload(j + _NB - 1)
        for j in range(max(0, NCH - _NB), NCH):
            if ho[j] is not None:
                ho[j].wait()
        plsc.subcore_barrier()
        return carry

    lax.fori_loop(0, V, _per_target, 0)


def kernel(features, xy_coords, extrinsics, W, b):
    assert features.shape == (1, V, G, C)
    assert xy_coords.shape == (1, V, V, G, 2)

    A, At = _adj_call(extrinsics.reshape(V, 16))

    y = _mm_call(features.reshape(VG, C), W, b.reshape(1, C))

    xc = jnp.moveaxis(xy_coords.reshape(V * V, G, 2), -1, 0)
    idxr, ow = _idx_call(xc[0], xc[1])
    idxr4 = idxr.reshape(2, V * V, G // 128, 128)
    ow4 = ow.reshape(2, V * V, G // 128, 128)

    at128 = jnp.zeros((128,), jnp.float32).at[:V * V].set(At.reshape(V * V))
    out2 = _sc_aggregate(y, idxr4, ow4, at128)
    return (out2[:VG].reshape(1, V, G, C), A.reshape(1, V, V))


# speedup vs baseline: 4.8055x; 1.0067x over previous
import functools

import jax
import jax.numpy as jnp
from jax import lax
from jax.experimental import pallas as pl
from jax.experimental.pallas import tpu as pltpu
from jax.experimental.pallas import tpu_sc as plsc

V = 8
G = 16384
C = 128
HW = 16384
VG = V * G
NS = 16
GP = G // NS
NCH = GP // 128
HHW = HW // 2
NDUM = 16
ODUM = 128


def _adj_body(e_ref, a_ref, at_ref):
    E = e_ref[...]
    cam = []
    for j in range(3):
        cj = -(E[:, 0 * 4 + j] * E[:, 3]
               + E[:, 1 * 4 + j] * E[:, 7]
               + E[:, 2 * 4 + j] * E[:, 11])
        cam.append(cj)
    dist_sq = ((cam[0][:, None] - cam[0][None, :]) ** 2
               + (cam[1][:, None] - cam[1][None, :]) ** 2
               + (cam[2][:, None] - cam[2][None, :]) ** 2)
    r_i = lax.broadcasted_iota(jnp.int32, (V, V), 0)
    c_i = lax.broadcasted_iota(jnp.int32, (V, V), 1)
    diag = r_i == c_i
    neg_inf = jnp.float32(-jnp.inf)
    work = jnp.where(diag, neg_inf, -dist_sq)
    mask = jnp.zeros((V, V), jnp.float32)
    for _ in range(3):
        m = jnp.max(work, axis=1, keepdims=True)
        eq = work == m
        cand = jnp.where(eq, c_i, V)
        jstar = jnp.min(cand, axis=1, keepdims=True)
        sel = c_i == jstar
        mask = jnp.where(sel, 1.0, mask)
        work = jnp.where(sel, neg_inf, work)
    a_overlap = 1.0 / (1.0 + jnp.sqrt(dist_sq + 1e-6))
    A = a_overlap * mask
    A = jnp.where(diag, 1.0, A)
    deg = jnp.sum(A, axis=1, keepdims=True)
    deg_inv = 1.0 / (deg + jnp.where(deg == 0, 1.0, 0.0))
    a_ref[...] = A
    at_ref[...] = A * deg_inv


_adj_call = pl.pallas_call(
    _adj_body,
    out_shape=(jax.ShapeDtypeStruct((V, V), jnp.float32),
               jax.ShapeDtypeStruct((V, V), jnp.float32)),
)


_RB = 2048


def _mm_body(x_ref, w_ref, b_ref, y_ref):
    y_ref[...] = (jnp.dot(x_ref[...], w_ref[...],
                          preferred_element_type=jnp.float32) + b_ref[...])


_mm_call = pl.pallas_call(
    _mm_body,
    grid=(VG // _RB,),
    in_specs=[
        pl.BlockSpec((_RB, C), lambda i: (i, 0)),
        pl.BlockSpec((C, C), lambda i: (0, 0)),
        pl.BlockSpec((1, C), lambda i: (0, 0)),
    ],
    out_specs=pl.BlockSpec((_RB, C), lambda i: (i, 0)),
    out_shape=jax.ShapeDtypeStruct((VG, C), jnp.float32),
)


def _idx_body(x_ref, y_ref, idxr_ref, ow_ref):
    xi = jnp.clip(jnp.round(x_ref[...]), 0, 127).astype(jnp.int32)
    yi = jnp.clip(jnp.round(y_ref[...]), 0, 127).astype(jnp.int32)
    idx = yi * 128 + xi
    i_row = lax.broadcasted_iota(jnp.int32, (V, G), 0)
    g_col = lax.broadcasted_iota(jnp.int32, (V, G), 1)
    grid_dum = HHW + (g_col & (NDUM - 1))
    out_dum = VG + (g_col & (ODUM - 1))
    out_real = i_row * G + g_col
    for c in range(2):
        t = idx - c * HHW
        inr = jnp.logical_and(t >= 0, t < HHW)
        idxr_ref[c] = jnp.where(inr, t, grid_dum)
        ow_ref[c] = jnp.where(inr, out_real, out_dum)


_idx_call = pl.pallas_call(
    _idx_body,
    grid=(8,),
    in_specs=[pl.BlockSpec((8, G), lambda i: (i, 0)),
              pl.BlockSpec((8, G), lambda i: (i, 0))],
    out_specs=[pl.BlockSpec((2, 8, G), lambda i: (0, i, 0)),
               pl.BlockSpec((2, 8, G), lambda i: (0, i, 0))],
    out_shape=[jax.ShapeDtypeStruct((2, V * V, G), jnp.int32),
               jax.ShapeDtypeStruct((2, V * V, G), jnp.int32)],
)


_mesh = plsc.VectorSubcoreMesh(core_axis_name="c", subcore_axis_name="s")


_NB = 3
_ZR = 32


@functools.partial(
    pl.kernel,
    mesh=_mesh,
    out_type=jax.ShapeDtypeStruct((VG + ODUM, C), jnp.float32),
    scratch_types=[
        pltpu.VMEM_SHARED((HHW + NDUM, C), jnp.float32),
        pltpu.VMEM((NCH, 128), jnp.int32),
        pltpu.VMEM((NCH, 128), jnp.int32),
        pltpu.VMEM((NCH, 128), jnp.int32),
        pltpu.VMEM((_NB, 128, C), jnp.float32),
        pltpu.VMEM((_ZR, C), jnp.float32),
        pltpu.VMEM((128,), jnp.float32),
    ] + [pltpu.SemaphoreType.DMA] * (2 * _NB),
)
def _sc_aggregate(y_hbm, idxr_hbm, ow_hbm, at_hbm, out_hbm,
                  grid_sh, idxbuf, selfbuf, owbuf, buf, zbuf, wbuf, *sems):
    c = lax.axis_index("c")
    s = lax.axis_index("s")
    sem_l = sems[:_NB]
    sem_s = sems[_NB:]
    pltpu.sync_copy(at_hbm, wbuf)

    zero16 = jnp.zeros((16,), jnp.float32)

    def _zrow(r, carry):
        for q in range(C // 16):
            zbuf[r, pl.ds(q * 16, 16)] = zero16
        return carry

    lax.fori_loop(0, _ZR, _zrow, 0)

    def _per_target(i, carry):
        nz = HHW // (_ZR * NS)
        hz = [None] * nz
        for j in range(nz):
            hz[j] = pltpu.async_copy(
                zbuf, grid_sh.at[pl.ds(s * (HHW // NS) + j * _ZR, _ZR), :],
                sem_l[j % _NB])
            if j >= _NB:
                pass
        for j in range(nz):
            hz[j].wait()
        plsc.subcore_barrier()

        def _per_view(v, cc):
            w = wbuf[pl.ds(i * V + v, 16)][0]

            @pl.when(w != 0.0)
            def _():
                pltpu.sync_copy(
                    idxr_hbm.at[c, v * V + i, pl.ds(s * NCH, NCH), :], idxbuf)

                def _load(j):
                    row0 = (v * 128 + s * NCH + j) * 128
                    return pltpu.async_copy(
                        y_hbm.at[pl.ds(row0, 128), :], buf.at[j % _NB],
                        sem_l[j % _NB])

                hl = [None] * NCH
                hs = [None] * NCH
                for j in range(_NB - 1):
                    hl[j] = _load(j)
                for j in range(NCH):
                    b = j % _NB
                    hl[j].wait()

                    @plsc.parallel_loop(0, 128, unroll=4)
                    def _scale(r):
                        for q in range(C // 16):
                            sl = pl.ds(q * 16, 16)
                            buf[b, r, sl] = buf[b, r, sl] * w
                    hs[j] = pltpu.async_copy(
                        buf.at[b], grid_sh.at[idxbuf.at[j]], sem_s[b], add=True)
                    if j + _NB - 1 < NCH:
                        if j >= 1:
                            hs[j - 1].wait()
                        hl[j + _NB - 1] = _load(j + _NB - 1)
                for j in range(max(0, NCH - _NB), NCH):
                    if hs[j] is not None:
                        hs[j].wait()

            return cc

        lax.fori_loop(0, V, _per_view, 0)
        plsc.subcore_barrier()

        pltpu.sync_copy(idxr_hbm.at[c, i * V + i, pl.ds(s * NCH, NCH), :], selfbuf)
        pltpu.sync_copy(ow_hbm.at[c, i * V + i, pl.ds(s * NCH, NCH), :], owbuf)

        def _gload(j):
            return pltpu.async_copy(
                grid_sh.at[selfbuf.at[j]], buf.at[j % _NB], sem_l[j % _NB])

        hg = [None] * NCH
        ho = [None] * NCH
        for j in range(_NB - 1):
            hg[j] = _gload(j)
        for j in range(NCH):
            b = j % _NB
            hg[j].wait()
            ho[j] = pltpu.async_copy(buf.at[b], out_hbm.at[owbuf.at[j]], sem_s[b])
            if j + _NB - 1 < NCH:
                if j >= 1:
                    ho[j - 1].wait()
                hg[j + _NB - 1] = _gload(j + _NB - 1)
        for j in range(max(0, NCH - _NB), NCH):
            if ho[j] is not None:
                ho[j].wait()
        plsc.subcore_barrier()
        return carry

    lax.fori_loop(0, V, _per_target, 0)


def kernel(features, xy_coords, extrinsics, W, b):
    assert features.shape == (1, V, G, C)
    assert xy_coords.shape == (1, V, V, G, 2)

    A, At = _adj_call(extrinsics.reshape(V, 16))

    y = _mm_call(features.reshape(VG, C), W, b.reshape(1, C))

    xc = jnp.moveaxis(xy_coords.reshape(V * V, G, 2), -1, 0)
    idxr, ow = _idx_call(xc[0], xc[1])
    idxr4 = idxr.reshape(2, V * V, G // 128, 128)
    ow4 = ow.reshape(2, V * V, G // 128, 128)

    at128 = jnp.zeros((128,), jnp.float32).at[:V * V].set(At.reshape(V * V))
    out2 = _sc_aggregate(y, idxr4, ow4, at128)
    return (out2[:VG].reshape(1, V, G, C), A.reshape(1, V, V))


# speedup vs baseline: 5.2395x; 1.0903x over previous
import functools

import jax
import jax.numpy as jnp
from jax import lax
from jax.experimental import pallas as pl
from jax.experimental.pallas import tpu as pltpu
from jax.experimental.pallas import tpu_sc as plsc

V = 8
G = 16384
C = 128
HW = 16384
VG = V * G
NS = 16
GP = G // NS
NCH = GP // 128
HHW = HW // 2
NDUM = 16
ODUM = 128


def _adj_body(e_ref, a_ref, at_ref, av_ref, wl_ref):
    E = e_ref[...]
    cam = []
    for j in range(3):
        cj = -(E[:, 0 * 4 + j] * E[:, 3]
               + E[:, 1 * 4 + j] * E[:, 7]
               + E[:, 2 * 4 + j] * E[:, 11])
        cam.append(cj)
    dist_sq = ((cam[0][:, None] - cam[0][None, :]) ** 2
               + (cam[1][:, None] - cam[1][None, :]) ** 2
               + (cam[2][:, None] - cam[2][None, :]) ** 2)
    r_i = lax.broadcasted_iota(jnp.int32, (V, V), 0)
    c_i = lax.broadcasted_iota(jnp.int32, (V, V), 1)
    diag = r_i == c_i
    neg_inf = jnp.float32(-jnp.inf)
    work = jnp.where(diag, neg_inf, -dist_sq)
    mask = jnp.zeros((V, V), jnp.float32)
    for _ in range(3):
        m = jnp.max(work, axis=1, keepdims=True)
        eq = work == m
        cand = jnp.where(eq, c_i, V)
        jstar = jnp.min(cand, axis=1, keepdims=True)
        sel = c_i == jstar
        mask = jnp.where(sel, 1.0, mask)
        work = jnp.where(sel, neg_inf, work)
    a_overlap = 1.0 / (1.0 + jnp.sqrt(dist_sq + 1e-6))
    A = a_overlap * mask
    A = jnp.where(diag, 1.0, A)
    deg = jnp.sum(A, axis=1, keepdims=True)
    deg_inv = 1.0 / (deg + jnp.where(deg == 0, 1.0, 0.0))
    At = A * deg_inv
    a_ref[...] = A
    at_ref[...] = At
    active = At != 0.0
    tri = (r_i <= c_i).astype(jnp.float32)
    rank = jnp.dot(active.astype(jnp.float32), tri,
                   preferred_element_type=jnp.float32).astype(jnp.int32) - 1
    av = jnp.zeros((V, V), jnp.int32)
    wl = jnp.zeros((V, V), jnp.float32)
    for r in range(V):
        hit = jnp.logical_and(active, rank == r)
        col_v = jnp.sum(jnp.where(hit, c_i, 0), axis=1, keepdims=True)
        col_w = jnp.sum(jnp.where(hit, At, 0.0), axis=1, keepdims=True)
        av = jnp.where(c_i == r, col_v, av)
        wl = jnp.where(c_i == r, col_w, wl)
    av_ref[...] = av
    wl_ref[...] = wl


_adj_call = pl.pallas_call(
    _adj_body,
    out_shape=(jax.ShapeDtypeStruct((V, V), jnp.float32),
               jax.ShapeDtypeStruct((V, V), jnp.float32),
               jax.ShapeDtypeStruct((V, V), jnp.int32),
               jax.ShapeDtypeStruct((V, V), jnp.float32)),
)


_RB = 2048


def _mm_body(x_ref, w_ref, b_ref, y_ref):
    y_ref[...] = (jnp.dot(x_ref[...], w_ref[...],
                          preferred_element_type=jnp.float32) + b_ref[...])


_mm_call = pl.pallas_call(
    _mm_body,
    grid=(VG // _RB,),
    in_specs=[
        pl.BlockSpec((_RB, C), lambda i: (i, 0)),
        pl.BlockSpec((C, C), lambda i: (0, 0)),
        pl.BlockSpec((1, C), lambda i: (0, 0)),
    ],
    out_specs=pl.BlockSpec((_RB, C), lambda i: (i, 0)),
    out_shape=jax.ShapeDtypeStruct((VG, C), jnp.float32),
)


def _idx_body(x_ref, y_ref, idxr_ref, ow_ref):
    xi = jnp.clip(jnp.round(x_ref[...]), 0, 127).astype(jnp.int32)
    yi = jnp.clip(jnp.round(y_ref[...]), 0, 127).astype(jnp.int32)
    idx = yi * 128 + xi
    i_row = lax.broadcasted_iota(jnp.int32, (V, G), 0)
    g_col = lax.broadcasted_iota(jnp.int32, (V, G), 1)
    grid_dum = HHW + (g_col & (NDUM - 1))
    out_dum = VG + (g_col & (ODUM - 1))
    out_real = i_row * G + g_col
    for c in range(2):
        t = idx - c * HHW
        inr = jnp.logical_and(t >= 0, t < HHW)
        idxr_ref[c] = jnp.where(inr, t, grid_dum)
        ow_ref[c] = jnp.where(inr, out_real, out_dum)


_idx_call = pl.pallas_call(
    _idx_body,
    grid=(8,),
    in_specs=[pl.BlockSpec((8, G), lambda i: (i, 0)),
              pl.BlockSpec((8, G), lambda i: (i, 0))],
    out_specs=[pl.BlockSpec((2, 8, G), lambda i: (0, i, 0)),
               pl.BlockSpec((2, 8, G), lambda i: (0, i, 0))],
    out_shape=[jax.ShapeDtypeStruct((2, V * V, G), jnp.int32),
               jax.ShapeDtypeStruct((2, V * V, G), jnp.int32)],
)


_mesh = plsc.VectorSubcoreMesh(core_axis_name="c", subcore_axis_name="s")


_NB = 3
_ZR = 32


@functools.partial(
    pl.kernel,
    mesh=_mesh,
    out_type=jax.ShapeDtypeStruct((VG + ODUM, C), jnp.float32),
    scratch_types=[
        pltpu.VMEM_SHARED((HHW + NDUM, C), jnp.float32),
        pltpu.VMEM((2, NCH, 128), jnp.int32),
        pltpu.VMEM((NCH, 128), jnp.int32),
        pltpu.VMEM((NCH, 128), jnp.int32),
        pltpu.VMEM((_NB, 128, C), jnp.float32),
        pltpu.VMEM((_ZR, C), jnp.float32),
        pltpu.VMEM((128,), jnp.int32),
        pltpu.VMEM((128,), jnp.float32),
    ] + [pltpu.SemaphoreType.DMA] * (2 * _NB),
)
def _sc_aggregate(y_hbm, idxr_hbm, ow_hbm, av_hbm, wl_hbm, out_hbm,
                  grid_sh, idxbuf, selfbuf, owbuf, buf, zbuf, avbuf, wbuf, *sems):
    c = lax.axis_index("c")
    s = lax.axis_index("s")
    sem_l = sems[:_NB]
    sem_s = sems[_NB:]
    pltpu.sync_copy(av_hbm, avbuf)
    pltpu.sync_copy(wl_hbm, wbuf)

    zero16 = jnp.zeros((16,), jnp.float32)

    def _zrow(r, carry):
        for q in range(C // 16):
            zbuf[r, pl.ds(q * 16, 16)] = zero16
        return carry

    lax.fori_loop(0, _ZR, _zrow, 0)

    def _per_target(i, carry):
        nz = HHW // (_ZR * NS)
        hz = [None] * nz
        for j in range(nz):
            hz[j] = pltpu.async_copy(
                zbuf, grid_sh.at[pl.ds(s * (HHW // NS) + j * _ZR, _ZR), :],
                sem_l[j % _NB])
            if j >= _NB:
                pass
        for j in range(nz):
            hz[j].wait()
        plsc.subcore_barrier()

        NV = 4
        NT = NV * NCH
        vs = [avbuf[pl.ds(i * V + k, 16)][0] for k in range(NV)]
        ws = [wbuf[pl.ds(i * V + k, 16)][0] for k in range(NV)]
        pltpu.sync_copy(
            idxr_hbm.at[c, vs[0] * V + i, pl.ds(s * NCH, NCH), :],
            idxbuf.at[0])

        def _load(t):
            v = vs[t // NCH]
            row0 = (v * 128 + s * NCH + (t % NCH)) * 128
            return pltpu.async_copy(
                y_hbm.at[pl.ds(row0, 128), :], buf.at[t % _NB], sem_l[t % _NB])

        hl = [None] * NT
        last_scat = [None] * _NB
        for t in range(_NB - 1):
            hl[t] = _load(t)
        for t in range(NT):
            b = t % _NB
            k = t // NCH
            j = t % NCH
            if j == 3 and k + 1 < NV:
                pltpu.sync_copy(
                    idxr_hbm.at[c, vs[k + 1] * V + i, pl.ds(s * NCH, NCH), :],
                    idxbuf.at[(k + 1) % 2])
            hl[t].wait()
            w = ws[k]

            @plsc.parallel_loop(0, 128, unroll=4)
            def _scale(r):
                for q in range(C // 16):
                    sl = pl.ds(q * 16, 16)
                    buf[b, r, sl] = buf[b, r, sl] * w

            last_scat[b] = pltpu.async_copy(
                buf.at[b], grid_sh.at[idxbuf.at[k % 2, j]], sem_s[b], add=True)
            if t + _NB - 1 < NT:
                nb = (t + _NB - 1) % _NB
                if last_scat[nb] is not None:
                    last_scat[nb].wait()
                    last_scat[nb] = None
                hl[t + _NB - 1] = _load(t + _NB - 1)
        for b in range(_NB):
            if last_scat[b] is not None:
                last_scat[b].wait()
        plsc.subcore_barrier()

        pltpu.sync_copy(idxr_hbm.at[c, i * V + i, pl.ds(s * NCH, NCH), :], selfbuf)
        pltpu.sync_copy(ow_hbm.at[c, i * V + i, pl.ds(s * NCH, NCH), :], owbuf)

        def _gload(j):
            return pltpu.async_copy(
                grid_sh.at[selfbuf.at[j]], buf.at[j % _NB], sem_l[j % _NB])

        hg = [None] * NCH
        ho = [None] * NCH
        for j in range(_NB - 1):
            hg[j] = _gload(j)
        for j in range(NCH):
            b = j % _NB
            hg[j].wait()
            ho[j] = pltpu.async_copy(buf.at[b], out_hbm.at[owbuf.at[j]], sem_s[b])
            if j + _NB - 1 < NCH:
                if j >= 1:
                    ho[j - 1].wait()
                hg[j + _NB - 1] = _gload(j + _NB - 1)
        for j in range(max(0, NCH - _NB), NCH):
            if ho[j] is not None:
                ho[j].wait()
        plsc.subcore_barrier()
        return carry

    lax.fori_loop(0, V, _per_target, 0)


def kernel(features, xy_coords, extrinsics, W, b):
    assert features.shape == (1, V, G, C)
    assert xy_coords.shape == (1, V, V, G, 2)

    A, At, Av, Wl = _adj_call(extrinsics.reshape(V, 16))

    y = _mm_call(features.reshape(VG, C), W, b.reshape(1, C))

    xc = jnp.moveaxis(xy_coords.reshape(V * V, G, 2), -1, 0)
    idxr, ow = _idx_call(xc[0], xc[1])
    idxr4 = idxr.reshape(2, V * V, G // 128, 128)
    ow4 = ow.reshape(2, V * V, G // 128, 128)

    av128 = jnp.zeros((128,), jnp.int32).at[:V * V].set(Av.reshape(V * V))
    wl128 = jnp.zeros((128,), jnp.float32).at[:V * V].set(Wl.reshape(V * V))
    out2 = _sc_aggregate(y, idxr4, ow4, av128, wl128)
    return (out2[:VG].reshape(1, V, G, C), A.reshape(1, V, V))
